# baseline (speedup 1.0000x reference)
import jax
import jax.numpy as jnp
from jax import lax
from jax.experimental import pallas as pl
from jax.experimental.pallas import tpu as pltpu
from jax.experimental.pallas import tpu_sc as plsc

BATCH = 128
NF = 32768

NC = 2
NS = 16
NW = NC * NS
SC_ROWS = 32
SC_ROW0 = BATCH - SC_ROWS
CHUNK = 16384
NCHUNK = NF // CHUNK
LANES = 16
UNROLL = 8
ITERS = CHUNK // (LANES * UNROLL)

_F32_BIG = 3.4e38


def _sc_body(in_hbm, prev_hbm, out_hbm, buf0, buf1, pv_all, res_buf,
             sem0, sem1):
    wid = lax.axis_index("s") * NC + lax.axis_index("c")
    row = SC_ROW0 + wid

    pltpu.sync_copy(prev_hbm, pv_all)
    pv = plsc.load_gather(pv_all, [jnp.full((LANES,), row, jnp.int32)])

    d0 = pltpu.async_copy(in_hbm.at[row, pl.ds(0, CHUNK)], buf0, sem0)
    d1 = pltpu.async_copy(in_hbm.at[row, pl.ds(CHUNK, CHUNK)], buf1, sem1)

    def scan_chunk(buf, mind, minv):
        def body(i, carry):
            acc = list(carry)
            for k in range(UNROLL):
                v = buf[pl.ds(i * (LANES * UNROLL) + k * LANES, LANES)]
                d = jnp.abs(v - pv)
                md, mv = acc[k], acc[UNROLL + k]
                pred = d < md
                acc[k] = jnp.where(pred, d, md)
                acc[UNROLL + k] = jnp.where(pred, v, mv)
            return tuple(acc)

        res = lax.fori_loop(0, ITERS, body, tuple(mind) + tuple(minv))
        return list(res[:UNROLL]), list(res[UNROLL:])

    mind = [jnp.full((LANES,), _F32_BIG, jnp.float32)] * UNROLL
    minv = [jnp.zeros((LANES,), jnp.float32)] * UNROLL
    d0.wait()
    mind, minv = scan_chunk(buf0, mind, minv)
    d1.wait()
    mind, minv = scan_chunk(buf1, mind, minv)

    n = UNROLL
    while n > 1:
        n //= 2
        for k in range(n):
            pred = mind[k + n] < mind[k]
            mind[k] = jnp.where(pred, mind[k + n], mind[k])
            minv[k] = jnp.where(pred, minv[k + n], minv[k])
    _, vs = plsc.sort_key_val(mind[0], minv[0])
    res_buf[...] = vs
    pltpu.sync_copy(res_buf, out_hbm.at[wid])


def _sc_call(inp, prev_flat):
    mesh = plsc.VectorSubcoreMesh(core_axis_name="c", subcore_axis_name="s")
    f = pl.kernel(
        _sc_body,
        out_type=jax.ShapeDtypeStruct((SC_ROWS, LANES), jnp.float32),
        mesh=mesh,
        compiler_params=pltpu.CompilerParams(needs_layout_passes=False),
        scratch_types=[
            pltpu.VMEM((CHUNK,), jnp.float32),
            pltpu.VMEM((CHUNK,), jnp.float32),
            pltpu.VMEM((BATCH,), jnp.float32),
            pltpu.VMEM((LANES,), jnp.float32),
            pltpu.SemaphoreType.DMA,
            pltpu.SemaphoreType.DMA,
        ],
    )
    return f(inp, prev_flat)


RB = 32
_NACC = 4
_TW = 512


def _tc_body(x_ref, p_ref, o_ref):
    p = p_ref[...]
    accs = [
        jnp.full((RB, _TW), _F32_BIG, jnp.float32) for _ in range(_NACC)
    ]
    for j in range(NF // _TW):
        t = x_ref[:, j * _TW:(j + 1) * _TW] - p
        ti = lax.bitcast_convert_type(t, jnp.int32)
        a = ti & jnp.int32(0x7FFFFFFE)
        s = lax.shift_right_logical(ti, 31)
        e = lax.bitcast_convert_type(a | s, jnp.float32)
        accs[j % _NACC] = jnp.minimum(accs[j % _NACC], e)
    acc = jnp.minimum(
        jnp.minimum(accs[0], accs[1]), jnp.minimum(accs[2], accs[3])
    )
    m = jnp.min(acc, axis=1, keepdims=True)
    mi = lax.bitcast_convert_type(m, jnp.int32)
    d_rec = lax.bitcast_convert_type(mi & jnp.int32(-2), jnp.float32)
    val = p + jnp.where(mi & 1, -d_rec, d_rec)
    o_ref[...] = val


def _tc_call(inp, prev):
    return pl.pallas_call(
        _tc_body,
        grid=(SC_ROW0 // RB,),
        in_specs=[
            pl.BlockSpec((RB, NF), lambda i: (i, 0)),
            pl.BlockSpec((RB, 1), lambda i: (i, 0)),
        ],
        out_specs=pl.BlockSpec((RB, 1), lambda i: (i, 0)),
        out_shape=jax.ShapeDtypeStruct((SC_ROW0, 1), jnp.float32),
    )(inp, prev)


@jax.jit
def _closest(inp, prev):
    sc_out = _sc_call(inp, prev.reshape(BATCH))
    tc_out = _tc_call(inp, prev)
    return jnp.concatenate([tc_out, sc_out[:, :1]], axis=0)


def kernel(input, prev_output):
    return _closest(input, prev_output)


# speedup vs baseline: 2.6252x; 2.6252x over previous
import jax
import jax.numpy as jnp
from jax import lax
from jax.experimental import pallas as pl

BATCH = 128
NF = 32768
RB = 64
NACC = 4
TW = 512

_F32_BIG = 3.4e38


def _tc_body(x_ref, p_ref, o_ref):
    p = p_ref[...]
    accs = [jnp.full((RB, TW), _F32_BIG, jnp.float32) for _ in range(NACC)]
    for j in range(NF // TW):
        t = x_ref[:, j * TW:(j + 1) * TW] - p
        ti = lax.bitcast_convert_type(t, jnp.int32)
        a = ti & jnp.int32(0x7FFFFFFE)
        s = lax.shift_right_logical(ti, 31)
        e = lax.bitcast_convert_type(a | s, jnp.float32)
        accs[j % NACC] = jnp.minimum(accs[j % NACC], e)
    acc = jnp.minimum(
        jnp.minimum(accs[0], accs[1]), jnp.minimum(accs[2], accs[3])
    )
    m = jnp.min(acc, axis=1, keepdims=True)
    mi = lax.bitcast_convert_type(m, jnp.int32)
    d_rec = lax.bitcast_convert_type(mi & jnp.int32(-2), jnp.float32)
    val = p + jnp.where(mi & 1, -d_rec, d_rec)
    o_ref[...] = val


@jax.jit
def _closest(inp, prev):
    return pl.pallas_call(
        _tc_body,
        grid=(BATCH // RB,),
        in_specs=[
            pl.BlockSpec((RB, NF), lambda i: (i, 0)),
            pl.BlockSpec((RB, 1), lambda i: (i, 0)),
        ],
        out_specs=pl.BlockSpec((RB, 1), lambda i: (i, 0)),
        out_shape=jax.ShapeDtypeStruct((BATCH, 1), jnp.float32),
    )(inp, prev)


def kernel(input, prev_output):
    return _closest(input, prev_output)
